# Initial kernel scaffold; baseline (speedup 1.0000x reference)
import jax
import jax.numpy as jnp
from jax.experimental import pallas as pl


def kernel(img, depth, T, K):
    raise NotImplementedError("write your pallas kernel here")


# speedup vs baseline: 2.6727x; 2.6727x over previous
import functools

import jax
import jax.numpy as jnp
from jax import lax
from jax.experimental import pallas as pl
from jax.experimental.pallas import tpu as pltpu
from jax.experimental.pallas import tpu_sc as plsc

H = 512
W = 512
HW = H * W
NB = 4
NW = 32
PIX_PER_W = HW // NW
TILE = 2048
NTILES = HW // TILE
LANES = 16
BIG = 1e8


def _proj_body(ik_ref, t_ref, k_ref, d_ref, idx_ref, nvd_ref, wmin_ref, wmax_ref):
    def bf(z):
        return z.astype(jnp.bfloat16).astype(jnp.float32)

    d = d_ref[0]
    x = bf(lax.broadcasted_iota(jnp.int32, (H, W), 1).astype(jnp.float32))
    y = bf(lax.broadcasted_iota(jnp.int32, (H, W), 0).astype(jnp.float32))
    cx = (bf(ik_ref[0, 0]) * x + bf(ik_ref[0, 1]) * y + bf(ik_ref[0, 2])) * d
    cy = (bf(ik_ref[1, 0]) * x + bf(ik_ref[1, 1]) * y + bf(ik_ref[1, 2])) * d
    cz = (bf(ik_ref[2, 0]) * x + bf(ik_ref[2, 1]) * y + bf(ik_ref[2, 2])) * d
    cxq, cyq, czq = bf(cx), bf(cy), bf(cz)
    px = bf(t_ref[0, 0, 0]) * cxq + bf(t_ref[0, 0, 1]) * cyq + bf(t_ref[0, 0, 2]) * czq + bf(t_ref[0, 0, 3])
    py = bf(t_ref[0, 1, 0]) * cxq + bf(t_ref[0, 1, 1]) * cyq + bf(t_ref[0, 1, 2]) * czq + bf(t_ref[0, 1, 3])
    pz = bf(t_ref[0, 2, 0]) * cxq + bf(t_ref[0, 2, 1]) * cyq + bf(t_ref[0, 2, 2]) * czq + bf(t_ref[0, 2, 3])
    pxq, pyq, pzq = bf(px), bf(py), bf(pz)
    prx = bf(k_ref[0, 0]) * pxq + bf(k_ref[0, 1]) * pyq + bf(k_ref[0, 2]) * pzq + bf(k_ref[0, 3])
    pry = bf(k_ref[1, 0]) * pxq + bf(k_ref[1, 1]) * pyq + bf(k_ref[1, 2]) * pzq + bf(k_ref[1, 3])
    prz = bf(k_ref[2, 0]) * pxq + bf(k_ref[2, 1]) * pyq + bf(k_ref[2, 2]) * pzq + bf(k_ref[2, 3])
    u = prx / (prz + 1e-7)
    v = pry / (prz + 1e-7)
    uu = jnp.clip(jnp.round(u), 0.0, W - 1.0)
    vv = jnp.clip(jnp.round(v), 0.0, H - 1.0)
    valid = pz < 1e6
    idx = jnp.where(valid, vv * W + uu, jnp.float32(HW))
    idxi = idx.astype(jnp.int32)
    idx_ref[0] = idxi
    nvd_ref[0] = pz
    wid = jnp.right_shift(idxi, 13)
    ok = idxi < HW
    wlo = jnp.where(ok, wid, 63).reshape(H, W // LANES, LANES)
    whi = jnp.where(ok, wid, -1).reshape(H, W // LANES, LANES)
    wmin_ref[0] = jnp.min(wlo, axis=-1)
    wmax_ref[0] = jnp.max(whi, axis=-1)


def _project(invK, T, K, depth):
    return pl.pallas_call(
        _proj_body,
        grid=(NB,),
        in_specs=[
            pl.BlockSpec((4, 4), lambda b: (0, 0)),
            pl.BlockSpec((1, 4, 4), lambda b: (b, 0, 0)),
            pl.BlockSpec((4, 4), lambda b: (0, 0)),
            pl.BlockSpec((1, H, W), lambda b: (b, 0, 0)),
        ],
        out_specs=[
            pl.BlockSpec((1, H, W), lambda b: (b, 0, 0)),
            pl.BlockSpec((1, H, W), lambda b: (b, 0, 0)),
            pl.BlockSpec((1, H, W // LANES), lambda b: (b, 0, 0)),
            pl.BlockSpec((1, H, W // LANES), lambda b: (b, 0, 0)),
        ],
        out_shape=[
            jax.ShapeDtypeStruct((NB, H, W), jnp.int32),
            jax.ShapeDtypeStruct((NB, H, W), jnp.float32),
            jax.ShapeDtypeStruct((NB, H, W // LANES), jnp.int32),
            jax.ShapeDtypeStruct((NB, H, W // LANES), jnp.int32),
        ],
    )(invK, T, K, depth)


def _zbuf_body(pidx, pd, pr, pg, pb, wmin, wmax,
               dep_o, r_o, g_o, b_o,
               dep_v, r_v, g_v, b_v, win_v, idx_t, d_t, r_t, g_t, b_t, wmin_t, wmax_t):
    wid = lax.axis_index("s") * 2 + lax.axis_index("c")
    base = wid * PIX_PER_W
    fill = jnp.full((LANES,), BIG, jnp.float32)
    zero = jnp.zeros((LANES,), jnp.float32)
    pbig = jnp.full((LANES,), HW, jnp.int32)
    nchunk = TILE // LANES

    def batch_body(b, _):
        def init_body(i, _):
            s = i * LANES
            dep_v[pl.ds(s, LANES)] = fill
            r_v[pl.ds(s, LANES)] = zero
            g_v[pl.ds(s, LANES)] = zero
            b_v[pl.ds(s, LANES)] = zero
            win_v[pl.ds(s, LANES)] = pbig
            return 0
        lax.fori_loop(0, PIX_PER_W // LANES, init_body, 0)

        def tile_body(t, _):
            off = t * TILE
            pltpu.sync_copy(pidx.at[b, pl.ds(off, TILE)], idx_t)
            pltpu.sync_copy(pd.at[b, pl.ds(off, TILE)], d_t)
            pltpu.sync_copy(pr.at[b, pl.ds(off, TILE)], r_t)
            pltpu.sync_copy(pg.at[b, pl.ds(off, TILE)], g_t)
            pltpu.sync_copy(pb.at[b, pl.ds(off, TILE)], b_t)
            pltpu.sync_copy(wmin.at[b, pl.ds(t * nchunk, nchunk)], wmin_t)
            pltpu.sync_copy(wmax.at[b, pl.ds(t * nchunk, nchunk)], wmax_t)

            def group_body(gi, _):
                wv_mn = wmin_t[pl.ds(gi * LANES, LANES)]
                wv_mx = wmax_t[pl.ds(gi * LANES, LANES)]
                for rr in range(LANES):
                    c = gi * LANES + rr
                    hit = (wv_mn[rr] <= wid) & (wv_mx[rr] >= wid)

                    @pl.when(hit)
                    def _(c=c):
                        s = c * LANES
                        pix = idx_t[pl.ds(s, LANES)]
                        loc = pix - base
                        inb = (loc >= 0) & (loc < PIX_PER_W)
                        dd = d_t[pl.ds(s, LANES)]
                        locc = jnp.clip(loc, 0, PIX_PER_W - 1)
                        pvec = lax.iota(jnp.int32, LANES) + (off + s)
                        m = inb
                        for _r in range(LANES):
                            cur = plsc.load_gather(dep_v, [locc], mask=m)
                            wcur = plsc.load_gather(win_v, [locc], mask=m)
                            m = m & ((dd < cur) | ((dd == cur) & (pvec < wcur)))
                            plsc.store_scatter(dep_v, [locc], dd, mask=m)
                            plsc.store_scatter(win_v, [locc], pvec, mask=m)
                        fin = plsc.load_gather(dep_v, [locc], mask=inb)
                        wpt = plsc.load_gather(win_v, [locc], mask=inb)
                        wfin = inb & (dd == fin) & (pvec == wpt)
                        plsc.store_scatter(r_v, [locc], r_t[pl.ds(s, LANES)], mask=wfin)
                        plsc.store_scatter(g_v, [locc], g_t[pl.ds(s, LANES)], mask=wfin)
                        plsc.store_scatter(b_v, [locc], b_t[pl.ds(s, LANES)], mask=wfin)
                return 0
            lax.fori_loop(0, nchunk // LANES, group_body, 0)
            return 0
        lax.fori_loop(0, NTILES, tile_body, 0)

        pltpu.sync_copy(dep_v, dep_o.at[b, pl.ds(base, PIX_PER_W)])
        pltpu.sync_copy(r_v, r_o.at[b, pl.ds(base, PIX_PER_W)])
        pltpu.sync_copy(g_v, g_o.at[b, pl.ds(base, PIX_PER_W)])
        pltpu.sync_copy(b_v, b_o.at[b, pl.ds(base, PIX_PER_W)])
        return 0
    lax.fori_loop(0, NB, batch_body, 0)


def _zbuffer(pidx, pd, pr, pg, pb, wmin, wmax):
    mesh = plsc.VectorSubcoreMesh(core_axis_name="c", subcore_axis_name="s")
    f = functools.partial(
        pl.kernel,
        mesh=mesh,
        compiler_params=pltpu.CompilerParams(needs_layout_passes=False),
        out_type=[
            jax.ShapeDtypeStruct((NB, HW), jnp.float32),
            jax.ShapeDtypeStruct((NB, HW), jnp.float32),
            jax.ShapeDtypeStruct((NB, HW), jnp.float32),
            jax.ShapeDtypeStruct((NB, HW), jnp.float32),
        ],
        scratch_types=[
            pltpu.VMEM((PIX_PER_W,), jnp.float32),
            pltpu.VMEM((PIX_PER_W,), jnp.float32),
            pltpu.VMEM((PIX_PER_W,), jnp.float32),
            pltpu.VMEM((PIX_PER_W,), jnp.float32),
            pltpu.VMEM((PIX_PER_W,), jnp.int32),
            pltpu.VMEM((TILE,), jnp.int32),
            pltpu.VMEM((TILE,), jnp.float32),
            pltpu.VMEM((TILE,), jnp.float32),
            pltpu.VMEM((TILE,), jnp.float32),
            pltpu.VMEM((TILE,), jnp.float32),
            pltpu.VMEM((TILE // LANES,), jnp.int32),
            pltpu.VMEM((TILE // LANES,), jnp.int32),
        ],
    )(_zbuf_body)
    return f(pidx, pd, pr, pg, pb, wmin, wmax)


def _merge_body(r_ref, g_ref, b_ref, d_ref, rgb_o, dep_o, mask_o):
    r = r_ref[0]
    g = g_ref[0]
    bl = b_ref[0]
    d = d_ref[0]
    m = (d == 0.0).astype(jnp.float32)

    lo = W // 4
    hi = W // 4 * 3
    rowi = lax.broadcasted_iota(jnp.int32, (H, hi - lo), 0)
    colj = lax.broadcasted_iota(jnp.int32, (H, hi - lo), 1)
    interior_r = rowi >= 1
    interior_c = colj >= 1
    interior_rc = interior_r & interior_c

    def variants(x):
        x0 = x[:, lo:hi]
        xr = jnp.concatenate([x0[0:1, :], x0[:-1, :]], axis=0)
        xc = jnp.concatenate([x0[:, 0:1], x0[:, :-1]], axis=1)
        xrc_full = jnp.concatenate([xr[:, 0:1], xr[:, :-1]], axis=1)
        x1 = jnp.where(interior_r, xr, x0)
        x2 = jnp.where(interior_c, xc, x0)
        x3 = jnp.where(interior_rc, xrc_full, x0)
        return x0, x1, x2, x3

    d0, d1, d2, d3 = variants(d)
    dm = jnp.minimum(jnp.minimum(d0, d1), jnp.minimum(d2, d3))

    def choose(c0, c1, c2, c3):
        return jnp.where(d0 == dm, c0,
                         jnp.where(d1 == dm, c1,
                                   jnp.where(d2 == dm, c2, c3)))

    def merged(x):
        x0, x1, x2, x3 = variants(x)
        mid = choose(x0, x1, x2, x3)
        return jnp.concatenate([x[:, :lo], mid, x[:, hi:]], axis=1)

    rgb_o[0, 0] = merged(r)
    rgb_o[0, 1] = merged(g)
    rgb_o[0, 2] = merged(bl)
    dep_o[0, 0] = merged(d)
    mask_o[0, 0] = merged(m)


def _merge(r, g, b, d):
    spec = pl.BlockSpec((1, H, W), lambda i: (i, 0, 0))
    return pl.pallas_call(
        _merge_body,
        grid=(NB,),
        in_specs=[spec, spec, spec, spec],
        out_specs=[
            pl.BlockSpec((1, 3, H, W), lambda i: (i, 0, 0, 0)),
            pl.BlockSpec((1, 1, H, W), lambda i: (i, 0, 0, 0)),
            pl.BlockSpec((1, 1, H, W), lambda i: (i, 0, 0, 0)),
        ],
        out_shape=[
            jax.ShapeDtypeStruct((NB, 3, H, W), jnp.float32),
            jax.ShapeDtypeStruct((NB, 1, H, W), jnp.float32),
            jax.ShapeDtypeStruct((NB, 1, H, W), jnp.float32),
        ],
    )(r, g, b, d)


def kernel(img, depth, T, K):
    invK = jnp.linalg.inv(K)
    idx, nvd, wmin, wmax = _project(invK, T, K, depth.reshape(NB, H, W))
    pidx = idx.reshape(NB, HW)
    pd = nvd.reshape(NB, HW)
    imgf = img.reshape(NB, 3, HW)
    dep, r, g, b = _zbuffer(pidx, pd, imgf[:, 0], imgf[:, 1], imgf[:, 2],
                            wmin.reshape(NB, HW // LANES),
                            wmax.reshape(NB, HW // LANES))
    rgb, dep_out, mask = _merge(
        r.reshape(NB, H, W), g.reshape(NB, H, W), b.reshape(NB, H, W),
        dep.reshape(NB, H, W))
    return rgb, dep_out, mask


# speedup vs baseline: 3.2322x; 1.2093x over previous
import functools

import jax
import jax.numpy as jnp
from jax import lax
from jax.experimental import pallas as pl
from jax.experimental.pallas import tpu as pltpu
from jax.experimental.pallas import tpu_sc as plsc

H = 512
W = 512
HW = H * W
NB = 4
NW = 32
PIX_PER_W = HW // NW
TILE = 2048
NTILES = HW // TILE
LANES = 16
BIG = 1e8


def _proj_body(ik_ref, t_ref, k_ref, d_ref, idx_ref, nvd_ref, wmin_ref, wmax_ref):
    def bf(z):
        return z.astype(jnp.bfloat16).astype(jnp.float32)

    d = d_ref[0]
    x = bf(lax.broadcasted_iota(jnp.int32, (H, W), 1).astype(jnp.float32))
    y = bf(lax.broadcasted_iota(jnp.int32, (H, W), 0).astype(jnp.float32))
    cx = (bf(ik_ref[0, 0]) * x + bf(ik_ref[0, 1]) * y + bf(ik_ref[0, 2])) * d
    cy = (bf(ik_ref[1, 0]) * x + bf(ik_ref[1, 1]) * y + bf(ik_ref[1, 2])) * d
    cz = (bf(ik_ref[2, 0]) * x + bf(ik_ref[2, 1]) * y + bf(ik_ref[2, 2])) * d
    cxq, cyq, czq = bf(cx), bf(cy), bf(cz)
    px = bf(t_ref[0, 0, 0]) * cxq + bf(t_ref[0, 0, 1]) * cyq + bf(t_ref[0, 0, 2]) * czq + bf(t_ref[0, 0, 3])
    py = bf(t_ref[0, 1, 0]) * cxq + bf(t_ref[0, 1, 1]) * cyq + bf(t_ref[0, 1, 2]) * czq + bf(t_ref[0, 1, 3])
    pz = bf(t_ref[0, 2, 0]) * cxq + bf(t_ref[0, 2, 1]) * cyq + bf(t_ref[0, 2, 2]) * czq + bf(t_ref[0, 2, 3])
    pxq, pyq, pzq = bf(px), bf(py), bf(pz)
    prx = bf(k_ref[0, 0]) * pxq + bf(k_ref[0, 1]) * pyq + bf(k_ref[0, 2]) * pzq + bf(k_ref[0, 3])
    pry = bf(k_ref[1, 0]) * pxq + bf(k_ref[1, 1]) * pyq + bf(k_ref[1, 2]) * pzq + bf(k_ref[1, 3])
    prz = bf(k_ref[2, 0]) * pxq + bf(k_ref[2, 1]) * pyq + bf(k_ref[2, 2]) * pzq + bf(k_ref[2, 3])
    u = prx / (prz + 1e-7)
    v = pry / (prz + 1e-7)
    uu = jnp.clip(jnp.round(u), 0.0, W - 1.0)
    vv = jnp.clip(jnp.round(v), 0.0, H - 1.0)
    valid = pz < 1e6
    idx = jnp.where(valid, vv * W + uu, jnp.float32(HW))
    idxi = idx.astype(jnp.int32)
    idx_ref[0] = idxi
    nvd_ref[0] = pz
    wid = jnp.right_shift(idxi, 13)
    ok = idxi < HW
    wlo = jnp.where(ok, wid, 63).reshape(H, W // LANES, LANES)
    whi = jnp.where(ok, wid, -1).reshape(H, W // LANES, LANES)
    wmin_ref[0] = jnp.min(wlo, axis=-1)
    wmax_ref[0] = jnp.max(whi, axis=-1)


def _project(invK, T, K, depth):
    return pl.pallas_call(
        _proj_body,
        grid=(NB,),
        in_specs=[
            pl.BlockSpec((4, 4), lambda b: (0, 0)),
            pl.BlockSpec((1, 4, 4), lambda b: (b, 0, 0)),
            pl.BlockSpec((4, 4), lambda b: (0, 0)),
            pl.BlockSpec((1, H, W), lambda b: (b, 0, 0)),
        ],
        out_specs=[
            pl.BlockSpec((1, H, W), lambda b: (b, 0, 0)),
            pl.BlockSpec((1, H, W), lambda b: (b, 0, 0)),
            pl.BlockSpec((1, H, W // LANES), lambda b: (b, 0, 0)),
            pl.BlockSpec((1, H, W // LANES), lambda b: (b, 0, 0)),
        ],
        out_shape=[
            jax.ShapeDtypeStruct((NB, H, W), jnp.int32),
            jax.ShapeDtypeStruct((NB, H, W), jnp.float32),
            jax.ShapeDtypeStruct((NB, H, W // LANES), jnp.int32),
            jax.ShapeDtypeStruct((NB, H, W // LANES), jnp.int32),
        ],
    )(invK, T, K, depth)


def _zbuf_body(pidx, pd, pr, pg, pb, wmin, wmax,
               dep_o, r_o, g_o, b_o,
               dep_v, r_v, g_v, b_v, win_v, idx_t, d_t, r_t, g_t, b_t, wmin_t, wmax_t, sem):
    wid = lax.axis_index("s") * 2 + lax.axis_index("c")
    base = wid * PIX_PER_W
    fill = jnp.full((LANES,), BIG, jnp.float32)
    zero = jnp.zeros((LANES,), jnp.float32)
    pbig = jnp.full((LANES,), HW, jnp.int32)
    nchunk = TILE // LANES

    def batch_body(b, _):
        def init_body(i, _):
            s = i * LANES
            dep_v[pl.ds(s, LANES)] = fill
            r_v[pl.ds(s, LANES)] = zero
            g_v[pl.ds(s, LANES)] = zero
            b_v[pl.ds(s, LANES)] = zero
            win_v[pl.ds(s, LANES)] = pbig
            return 0
        lax.fori_loop(0, PIX_PER_W // LANES, init_body, 0)

        def tile_body(t, _):
            off = t * TILE
            cps = [
                pltpu.async_copy(pidx.at[b, pl.ds(off, TILE)], idx_t, sem),
                pltpu.async_copy(pd.at[b, pl.ds(off, TILE)], d_t, sem),
                pltpu.async_copy(pr.at[b, pl.ds(off, TILE)], r_t, sem),
                pltpu.async_copy(pg.at[b, pl.ds(off, TILE)], g_t, sem),
                pltpu.async_copy(pb.at[b, pl.ds(off, TILE)], b_t, sem),
                pltpu.async_copy(wmin.at[b, pl.ds(t * nchunk, nchunk)], wmin_t, sem),
                pltpu.async_copy(wmax.at[b, pl.ds(t * nchunk, nchunk)], wmax_t, sem),
            ]
            for cp in cps:
                cp.wait()

            def group_body(gi, _):
                wv_mn = wmin_t[pl.ds(gi * LANES, LANES)]
                wv_mx = wmax_t[pl.ds(gi * LANES, LANES)]
                for rr in range(LANES):
                    c = gi * LANES + rr
                    hit = (wv_mn[rr] <= wid) & (wv_mx[rr] >= wid)

                    @pl.when(hit)
                    def _(c=c):
                        s = c * LANES
                        pix = idx_t[pl.ds(s, LANES)]
                        loc = pix - base
                        inb = (loc >= 0) & (loc < PIX_PER_W)
                        dd = d_t[pl.ds(s, LANES)]
                        locc = jnp.clip(loc, 0, PIX_PER_W - 1)
                        pvec = lax.iota(jnp.int32, LANES) + (off + s)
                        m = inb
                        for _r in range(LANES):
                            cur = plsc.load_gather(dep_v, [locc], mask=m)
                            wcur = plsc.load_gather(win_v, [locc], mask=m)
                            m = m & ((dd < cur) | ((dd == cur) & (pvec < wcur)))
                            plsc.store_scatter(dep_v, [locc], dd, mask=m)
                            plsc.store_scatter(win_v, [locc], pvec, mask=m)
                        fin = plsc.load_gather(dep_v, [locc], mask=inb)
                        wpt = plsc.load_gather(win_v, [locc], mask=inb)
                        wfin = inb & (dd == fin) & (pvec == wpt)
                        plsc.store_scatter(r_v, [locc], r_t[pl.ds(s, LANES)], mask=wfin)
                        plsc.store_scatter(g_v, [locc], g_t[pl.ds(s, LANES)], mask=wfin)
                        plsc.store_scatter(b_v, [locc], b_t[pl.ds(s, LANES)], mask=wfin)
                return 0
            lax.fori_loop(0, nchunk // LANES, group_body, 0)
            return 0
        lax.fori_loop(0, NTILES, tile_body, 0)

        pltpu.sync_copy(dep_v, dep_o.at[b, pl.ds(base, PIX_PER_W)])
        pltpu.sync_copy(r_v, r_o.at[b, pl.ds(base, PIX_PER_W)])
        pltpu.sync_copy(g_v, g_o.at[b, pl.ds(base, PIX_PER_W)])
        pltpu.sync_copy(b_v, b_o.at[b, pl.ds(base, PIX_PER_W)])
        return 0
    lax.fori_loop(0, NB, batch_body, 0)


def _zbuffer(pidx, pd, pr, pg, pb, wmin, wmax):
    mesh = plsc.VectorSubcoreMesh(core_axis_name="c", subcore_axis_name="s")
    f = functools.partial(
        pl.kernel,
        mesh=mesh,
        compiler_params=pltpu.CompilerParams(needs_layout_passes=False),
        out_type=[
            jax.ShapeDtypeStruct((NB, HW), jnp.float32),
            jax.ShapeDtypeStruct((NB, HW), jnp.float32),
            jax.ShapeDtypeStruct((NB, HW), jnp.float32),
            jax.ShapeDtypeStruct((NB, HW), jnp.float32),
        ],
        scratch_types=[
            pltpu.VMEM((PIX_PER_W,), jnp.float32),
            pltpu.VMEM((PIX_PER_W,), jnp.float32),
            pltpu.VMEM((PIX_PER_W,), jnp.float32),
            pltpu.VMEM((PIX_PER_W,), jnp.float32),
            pltpu.VMEM((PIX_PER_W,), jnp.int32),
            pltpu.VMEM((TILE,), jnp.int32),
            pltpu.VMEM((TILE,), jnp.float32),
            pltpu.VMEM((TILE,), jnp.float32),
            pltpu.VMEM((TILE,), jnp.float32),
            pltpu.VMEM((TILE,), jnp.float32),
            pltpu.VMEM((TILE // LANES,), jnp.int32),
            pltpu.VMEM((TILE // LANES,), jnp.int32),
            pltpu.SemaphoreType.DMA,
        ],
    )(_zbuf_body)
    return f(pidx, pd, pr, pg, pb, wmin, wmax)


def _merge_body(r_ref, g_ref, b_ref, d_ref, rgb_o, dep_o, mask_o):
    r = r_ref[0]
    g = g_ref[0]
    bl = b_ref[0]
    d = d_ref[0]
    m = (d == 0.0).astype(jnp.float32)

    lo = W // 4
    hi = W // 4 * 3
    rowi = lax.broadcasted_iota(jnp.int32, (H, hi - lo), 0)
    colj = lax.broadcasted_iota(jnp.int32, (H, hi - lo), 1)
    interior_r = rowi >= 1
    interior_c = colj >= 1
    interior_rc = interior_r & interior_c

    def variants(x):
        x0 = x[:, lo:hi]
        xr = jnp.concatenate([x0[0:1, :], x0[:-1, :]], axis=0)
        xc = jnp.concatenate([x0[:, 0:1], x0[:, :-1]], axis=1)
        xrc_full = jnp.concatenate([xr[:, 0:1], xr[:, :-1]], axis=1)
        x1 = jnp.where(interior_r, xr, x0)
        x2 = jnp.where(interior_c, xc, x0)
        x3 = jnp.where(interior_rc, xrc_full, x0)
        return x0, x1, x2, x3

    d0, d1, d2, d3 = variants(d)
    dm = jnp.minimum(jnp.minimum(d0, d1), jnp.minimum(d2, d3))

    def choose(c0, c1, c2, c3):
        return jnp.where(d0 == dm, c0,
                         jnp.where(d1 == dm, c1,
                                   jnp.where(d2 == dm, c2, c3)))

    def merged(x):
        x0, x1, x2, x3 = variants(x)
        mid = choose(x0, x1, x2, x3)
        return jnp.concatenate([x[:, :lo], mid, x[:, hi:]], axis=1)

    rgb_o[0, 0] = merged(r)
    rgb_o[0, 1] = merged(g)
    rgb_o[0, 2] = merged(bl)
    dep_o[0, 0] = merged(d)
    mask_o[0, 0] = merged(m)


def _merge(r, g, b, d):
    spec = pl.BlockSpec((1, H, W), lambda i: (i, 0, 0))
    return pl.pallas_call(
        _merge_body,
        grid=(NB,),
        in_specs=[spec, spec, spec, spec],
        out_specs=[
            pl.BlockSpec((1, 3, H, W), lambda i: (i, 0, 0, 0)),
            pl.BlockSpec((1, 1, H, W), lambda i: (i, 0, 0, 0)),
            pl.BlockSpec((1, 1, H, W), lambda i: (i, 0, 0, 0)),
        ],
        out_shape=[
            jax.ShapeDtypeStruct((NB, 3, H, W), jnp.float32),
            jax.ShapeDtypeStruct((NB, 1, H, W), jnp.float32),
            jax.ShapeDtypeStruct((NB, 1, H, W), jnp.float32),
        ],
    )(r, g, b, d)


def kernel(img, depth, T, K):
    invK = jnp.linalg.inv(K)
    idx, nvd, wmin, wmax = _project(invK, T, K, depth.reshape(NB, H, W))
    pidx = idx.reshape(NB, HW)
    pd = nvd.reshape(NB, HW)
    imgf = img.reshape(NB, 3, HW)
    dep, r, g, b = _zbuffer(pidx, pd, imgf[:, 0], imgf[:, 1], imgf[:, 2],
                            wmin.reshape(NB, HW // LANES),
                            wmax.reshape(NB, HW // LANES))
    rgb, dep_out, mask = _merge(
        r.reshape(NB, H, W), g.reshape(NB, H, W), b.reshape(NB, H, W),
        dep.reshape(NB, H, W))
    return rgb, dep_out, mask
